# Initial kernel scaffold; baseline (speedup 1.0000x reference)
import jax
import jax.numpy as jnp
from jax.experimental import pallas as pl


def kernel(x, forward_shuffle_idx):
    raise NotImplementedError("write your pallas kernel here")


# speedup vs baseline: 2.7056x; 2.7056x over previous
import functools

import jax
import jax.numpy as jnp
from jax import lax
from jax.experimental import pallas as pl
from jax.experimental.pallas import tpu as pltpu
from jax.experimental.pallas import tpu_sc as plsc

_B, _T, _D = 4, 3072, 1024
_NC = 2
_NS = 16
_NW = _NC * _NS
_WPB = _NW // _B
_RPW = _T // _WPB
_NCHUNK = 8
_CHUNK = _RPW // _NCHUNK
_LANES = 16


@jax.jit
def _sc_permute(x2d, idx):
    mesh = plsc.VectorSubcoreMesh(core_axis_name="c", subcore_axis_name="s")

    @functools.partial(
        pl.kernel,
        out_type=jax.ShapeDtypeStruct((_B * _T, _D), jnp.float32),
        mesh=mesh,
        scratch_types=[
            pltpu.VMEM((_RPW,), jnp.int32),
            pltpu.VMEM((_NCHUNK, _CHUNK), jnp.int32),
            pltpu.VMEM((2, _CHUNK, _D), jnp.float32),
            pltpu.SemaphoreType.DMA,
            pltpu.SemaphoreType.DMA,
            pltpu.SemaphoreType.DMA,
            pltpu.SemaphoreType.DMA,
        ],
    )
    def k(x_hbm, idx_hbm, out_hbm, raw_v, gidx_v, buf_v,
          gsem0, gsem1, wsem0, wsem1):
        wid = lax.axis_index("s") * _NC + lax.axis_index("c")
        b = wid // _WPB
        tbase = (wid % _WPB) * _RPW
        obase = wid * _RPW
        boff = b * _T

        pltpu.sync_copy(idx_hbm.at[pl.ds(tbase, _RPW)], raw_v)
        for c in range(_NCHUNK):
            for kk in range(_CHUNK // _LANES):
                o = c * _CHUNK + kk * _LANES
                gidx_v[c, pl.ds(kk * _LANES, _LANES)] = (
                    raw_v[pl.ds(o, _LANES)] + boff
                )

        gsems = [gsem0, gsem1]
        wsems = [wsem0, wsem1]

        def start_gather(c):
            return pltpu.async_copy(
                x_hbm.at[gidx_v.at[c]], buf_v.at[c % 2], gsems[c % 2]
            )

        gh = [None] * _NCHUNK
        wh = [None] * _NCHUNK
        gh[0] = start_gather(0)
        gh[1] = start_gather(1)
        for c in range(_NCHUNK):
            p = c % 2
            gh[c].wait()
            wh[c] = pltpu.async_copy(
                buf_v.at[p],
                out_hbm.at[pl.ds(obase + c * _CHUNK, _CHUNK)],
                wsems[p],
            )
            if c + 2 < _NCHUNK:
                wh[c].wait()
                gh[c + 2] = start_gather(c + 2)
        wh[_NCHUNK - 2].wait()
        wh[_NCHUNK - 1].wait()

    return k(x2d, idx)


def kernel(x, forward_shuffle_idx):
    x2d = x.reshape(_B * _T, _D)
    out2d = _sc_permute(x2d, forward_shuffle_idx.astype(jnp.int32))
    return out2d.reshape(_B, _T, _D)
